# baseline (speedup 1.0000x reference)
import math

import jax
import jax.numpy as jnp
from jax.experimental import pallas as pl
from jax.experimental.pallas import tpu as pltpu

_B, _T, _DIM = 4, 4096, 1024
_S, _H = 8, 8
_HD = _DIM // _H
_L = _T // _S
_NJ = 2
_SPS = _S // _NJ
_RPS = _SPS * _L
_C = _S * _H


def _prep_kernel(wwk_ref, wwq_ref, ms_ref, wrq_ref, wrk_ref, wwv_ref,
                 wscore_ref, wqk_ref, wwv_out_ref):
    bf = jnp.bfloat16
    qwT = jax.lax.dot_general(wwq_ref[...].astype(bf), ms_ref[...].astype(bf),
                              (((1,), (1,)), ((), ())),
                              preferred_element_type=jnp.float32)
    ci = jax.lax.broadcasted_iota(jnp.int32, (_DIM, _C), 1)
    ri = jax.lax.broadcasted_iota(jnp.int32, (_DIM, _C), 0)
    hsel = (ci // _S) == (ri // _HD)
    g = jnp.zeros((_DIM, _C), jnp.float32)
    for s in range(_S):
        g = g + jnp.where(hsel & (ci % _S == s),
                          qwT[:, s:s + 1] * (1.0 / math.sqrt(_HD)), 0.0)
    wscore_ref[...] = jax.lax.dot_general(
        wwk_ref[...].astype(bf), g.astype(bf), (((0,), (0,)), ((), ())),
        preferred_element_type=jnp.float32).astype(bf)
    wqk_ref[...] = jax.lax.dot_general(
        wrq_ref[...].astype(bf), wrk_ref[...].astype(bf),
        (((0,), (0,)), ((), ())),
        preferred_element_type=jnp.float32).astype(bf)
    wwv_out_ref[...] = wwv_ref[...].astype(bf)


def _memory_kernel(x_ref, wscore_ref, wqk_ref, wwv_ref, wrvT_ref, wm8_ref,
                   out_ref, memoutT_ref, memT_scr, xprev_scr):
    jj = pl.program_id(1)
    bf = jnp.bfloat16

    @pl.when(jj == 0)
    def _():
        memT_scr[...] = jnp.zeros_like(memT_scr)

    base_r = (_NJ - jj) * _SPS
    memb = memT_scr[...].astype(bf)
    p = jnp.dot(wqk_ref[...], memb,
                preferred_element_type=jnp.float32)
    xp = xprev_scr[...]
    qk = jnp.dot(xp, p.astype(bf),
                 preferred_element_type=jnp.float32) * (1.0 / math.sqrt(_DIM))
    si = jax.lax.broadcasted_iota(jnp.int32, (_RPS, _S), 1)
    ti2 = jax.lax.broadcasted_iota(jnp.int32, (_RPS, _S), 0) // _L
    qk = jnp.where(si >= ti2 + base_r, qk, -1e30)
    mr = jnp.max(qk, axis=1, keepdims=True)
    er = jnp.exp(qk - mr)
    attn_r = (er / jnp.sum(er, axis=1, keepdims=True)).astype(bf)
    vr = jax.lax.dot_general(memb, wrvT_ref[...], (((0,), (0,)), ((), ())),
                             preferred_element_type=jnp.float32)
    out = jnp.dot(attn_r, vr.astype(bf),
                  preferred_element_type=jnp.float32)
    out_ref[0, 0] = out * jax.nn.sigmoid(out)

    base_w = (_NJ - 1 - jj) * _SPS
    xb = x_ref[0, 0].astype(bf)
    scores = jnp.dot(xb, wscore_ref[...],
                     preferred_element_type=jnp.float32)
    ti = jax.lax.broadcasted_iota(jnp.int32, (_RPS, _C), 0) // _L
    ci2 = jax.lax.broadcasted_iota(jnp.int32, (_RPS, _C), 1) % _S
    wmask = ci2 == (ti + base_w)
    scores = jnp.where(wmask, scores, -1e30)
    mw = jnp.max(scores, axis=0, keepdims=True)
    ew = jnp.exp(scores - mw)
    attn_w = ew / (jnp.sum(ew, axis=0, keepdims=True) + 1e-30)
    attn_w = jnp.where(wmask, attn_w, 0.0).astype(bf)

    pooled = jax.lax.dot_general(attn_w, xb, (((0,), (0,)), ((), ())),
                                 preferred_element_type=jnp.float32)
    mem2T = jax.lax.dot_general(wwv_ref[...], pooled.astype(bf),
                                (((1,), (1,)), ((), ())),
                                preferred_element_type=jnp.float32)
    row_head = jax.lax.broadcasted_iota(jnp.int32, (_DIM, _S), 0) // _HD
    memT_new = jnp.zeros((_DIM, _S), jnp.float32)
    for h in range(_H):
        memT_new = memT_new + jnp.where(row_head == h,
                                        mem2T[:, h * _S:(h + 1) * _S], 0.0)
    memT_new = memT_new * wm8_ref[...]
    var = jnp.sum(memT_new * memT_new, axis=0, keepdims=True) * (1.0 / _DIM)
    memT_new = memT_new * jax.lax.rsqrt(var + 1e-6)
    memT_scr[...] = memT_scr[...] + memT_new
    memoutT_ref[0] = memT_scr[...]
    xprev_scr[...] = xb


def _run(x, memory_slots, W_read_q, W_read_kv, W_write_q, W_write_k, W_write_v,
         write_matter, interpret=False):
    xr = x.reshape(_B, _NJ, _RPS, _DIM)
    bf = jnp.bfloat16
    wrvT = W_read_kv[_DIM:].T.astype(bf)
    wm8 = jnp.broadcast_to(write_matter[:, None], (_DIM, _S))

    wscore, wqk, wwv = pl.pallas_call(
        _prep_kernel,
        out_shape=[
            jax.ShapeDtypeStruct((_DIM, _C), bf),
            jax.ShapeDtypeStruct((_DIM, _DIM), bf),
            jax.ShapeDtypeStruct((_DIM, _DIM), bf),
        ],
        name="memory_prep",
        interpret=interpret,
    )(W_write_k, W_write_q, memory_slots, W_read_q, W_read_kv[:_DIM],
      W_write_v)

    out, memT = pl.pallas_call(
        _memory_kernel,
        grid=(_B, _NJ + 1),
        in_specs=[
            pl.BlockSpec((1, 1, _RPS, _DIM),
                         lambda b, j: (b, _NJ - 1 - jnp.minimum(j, _NJ - 1),
                                       0, 0)),
            pl.BlockSpec((_DIM, _C), lambda b, j: (0, 0)),
            pl.BlockSpec((_DIM, _DIM), lambda b, j: (0, 0)),
            pl.BlockSpec((_DIM, _DIM), lambda b, j: (0, 0)),
            pl.BlockSpec((_DIM, _DIM), lambda b, j: (0, 0)),
            pl.BlockSpec((_DIM, _S), lambda b, j: (0, 0)),
        ],
        out_specs=[
            pl.BlockSpec((1, 1, _RPS, _DIM),
                         lambda b, j: (b, _NJ - jnp.maximum(j, 1), 0, 0)),
            pl.BlockSpec((1, _DIM, _S), lambda b, j: (b, 0, 0)),
        ],
        out_shape=[
            jax.ShapeDtypeStruct((_B, _NJ, _RPS, _DIM), jnp.float32),
            jax.ShapeDtypeStruct((_B, _DIM, _S), jnp.float32),
        ],
        scratch_shapes=[
            pltpu.VMEM((_DIM, _S), jnp.float32),
            pltpu.VMEM((_RPS, _DIM), jnp.bfloat16),
        ],
        compiler_params=pltpu.CompilerParams(
            dimension_semantics=("parallel", "arbitrary"),
            vmem_limit_bytes=60000 * 1024,
        ),
        name="memory_fused",
        interpret=interpret,
    )(xr, wscore, wqk, wwv, wrvT, wm8)
    return out.reshape(_B, _T, _DIM), memT.transpose(0, 2, 1)


@jax.jit
def kernel(x, memory_slots, W_read_q, W_read_kv, W_write_q, W_write_k,
           W_write_v, write_matter):
    return _run(x, memory_slots, W_read_q, W_read_kv, W_write_q, W_write_k,
                W_write_v, write_matter)


# speedup vs baseline: 1.3210x; 1.3210x over previous
import math

import jax
import jax.numpy as jnp
from jax.experimental import pallas as pl
from jax.experimental.pallas import tpu as pltpu

_B, _T, _DIM = 4, 4096, 1024
_S, _H = 8, 8
_HD = _DIM // _H
_L = _T // _S
_NJ = 2
_SPS = _S // _NJ
_RPS = _SPS * _L
_C = _S * _H


def _prep_kernel(wwk_ref, wwq_ref, ms_ref, wrq_ref, wrk_ref, wwv_ref,
                 wscore_ref, wqk_ref, wwv_out_ref):
    bf = jnp.bfloat16
    qwT = jax.lax.dot_general(wwq_ref[...].astype(bf), ms_ref[...].astype(bf),
                              (((1,), (1,)), ((), ())),
                              preferred_element_type=jnp.float32)
    ci = jax.lax.broadcasted_iota(jnp.int32, (_DIM, _C), 1)
    ri = jax.lax.broadcasted_iota(jnp.int32, (_DIM, _C), 0)
    hsel = (ci // _S) == (ri // _HD)
    g = jnp.zeros((_DIM, _C), jnp.float32)
    for s in range(_S):
        g = g + jnp.where(hsel & (ci % _S == s),
                          qwT[:, s:s + 1] * (1.0 / math.sqrt(_HD)), 0.0)
    wscore_ref[...] = jax.lax.dot_general(
        wwk_ref[...].astype(bf), g.astype(bf), (((0,), (0,)), ((), ())),
        preferred_element_type=jnp.float32).astype(bf)
    wqk_ref[...] = jax.lax.dot_general(
        wrq_ref[...].astype(bf), wrk_ref[...].astype(bf),
        (((0,), (0,)), ((), ())),
        preferred_element_type=jnp.float32).astype(bf)
    wwv_out_ref[...] = wwv_ref[...].astype(bf)


def _memory_kernel(x_ref, wscore_ref, wqk_ref, wwv_ref, wrvT_ref, wm8_ref,
                   out_ref, memoutT_ref, memT_scr):
    j = pl.program_id(1)
    base = (_NJ - 1 - j) * _SPS
    bf = jnp.bfloat16

    @pl.when(j == 0)
    def _():
        memT_scr[...] = jnp.zeros_like(memT_scr)

    xb = x_ref[0, 0].astype(bf)

    scores = jnp.dot(xb, wscore_ref[...],
                     preferred_element_type=jnp.float32)
    ti = jax.lax.broadcasted_iota(jnp.int32, (_RPS, _C), 0) // _L
    ci2 = jax.lax.broadcasted_iota(jnp.int32, (_RPS, _C), 1) % _S
    wmask = ci2 == (ti + base)
    scores = jnp.where(wmask, scores, -1e30)
    mw = jnp.max(scores, axis=0, keepdims=True)
    ew = jnp.exp(scores - mw)
    attn_w = ew / (jnp.sum(ew, axis=0, keepdims=True) + 1e-30)
    attn_w = jnp.where(wmask, attn_w, 0.0).astype(bf)

    pooled = jax.lax.dot_general(attn_w, xb, (((0,), (0,)), ((), ())),
                                 preferred_element_type=jnp.float32)
    mem2T = jax.lax.dot_general(wwv_ref[...], pooled.astype(bf),
                                (((1,), (1,)), ((), ())),
                                preferred_element_type=jnp.float32)
    row_head = jax.lax.broadcasted_iota(jnp.int32, (_DIM, _S), 0) // _HD
    memT_new = jnp.zeros((_DIM, _S), jnp.float32)
    for h in range(_H):
        memT_new = memT_new + jnp.where(row_head == h,
                                        mem2T[:, h * _S:(h + 1) * _S], 0.0)
    memT_new = memT_new * wm8_ref[...]
    var = jnp.sum(memT_new * memT_new, axis=0, keepdims=True) * (1.0 / _DIM)
    memT_new = memT_new * jax.lax.rsqrt(var + 1e-6)
    memT_scr[...] = memT_scr[...] + memT_new
    memoutT_ref[0] = memT_scr[...]

    memb = memT_scr[...].astype(bf)
    p = jnp.dot(wqk_ref[...], memb,
                preferred_element_type=jnp.float32)
    qk = jnp.dot(xb, p.astype(bf),
                 preferred_element_type=jnp.float32) * (1.0 / math.sqrt(_DIM))
    si = jax.lax.broadcasted_iota(jnp.int32, (_RPS, _S), 1)
    ti2 = jax.lax.broadcasted_iota(jnp.int32, (_RPS, _S), 0) // _L
    qk = jnp.where(si >= ti2 + base, qk, -1e30)
    mr = jnp.max(qk, axis=1, keepdims=True)
    er = jnp.exp(qk - mr)
    attn_r = (er / jnp.sum(er, axis=1, keepdims=True)).astype(bf)
    vr = jax.lax.dot_general(memb, wrvT_ref[...], (((0,), (0,)), ((), ())),
                             preferred_element_type=jnp.float32)
    out = jnp.dot(attn_r, vr.astype(bf),
                  preferred_element_type=jnp.float32)
    out_ref[0, 0] = out * jax.nn.sigmoid(out)


def _run(x, memory_slots, W_read_q, W_read_kv, W_write_q, W_write_k, W_write_v,
         write_matter, interpret=False):
    xr = x.reshape(_B, _NJ, _RPS, _DIM)
    bf = jnp.bfloat16
    wrvT = W_read_kv[_DIM:].T.astype(bf)
    wm8 = jnp.broadcast_to(write_matter[:, None], (_DIM, _S))

    wscore, wqk, wwv = pl.pallas_call(
        _prep_kernel,
        out_shape=[
            jax.ShapeDtypeStruct((_DIM, _C), bf),
            jax.ShapeDtypeStruct((_DIM, _DIM), bf),
            jax.ShapeDtypeStruct((_DIM, _DIM), bf),
        ],
        name="memory_prep",
        interpret=interpret,
    )(W_write_k, W_write_q, memory_slots, W_read_q, W_read_kv[:_DIM],
      W_write_v)

    out, memT = pl.pallas_call(
        _memory_kernel,
        grid=(_B, _NJ),
        in_specs=[
            pl.BlockSpec((1, 1, _RPS, _DIM), lambda b, j: (b, _NJ - 1 - j, 0, 0)),
            pl.BlockSpec((_DIM, _C), lambda b, j: (0, 0)),
            pl.BlockSpec((_DIM, _DIM), lambda b, j: (0, 0)),
            pl.BlockSpec((_DIM, _DIM), lambda b, j: (0, 0)),
            pl.BlockSpec((_DIM, _DIM), lambda b, j: (0, 0)),
            pl.BlockSpec((_DIM, _S), lambda b, j: (0, 0)),
        ],
        out_specs=[
            pl.BlockSpec((1, 1, _RPS, _DIM), lambda b, j: (b, _NJ - 1 - j, 0, 0)),
            pl.BlockSpec((1, _DIM, _S), lambda b, j: (b, 0, 0)),
        ],
        out_shape=[
            jax.ShapeDtypeStruct((_B, _NJ, _RPS, _DIM), jnp.float32),
            jax.ShapeDtypeStruct((_B, _DIM, _S), jnp.float32),
        ],
        scratch_shapes=[
            pltpu.VMEM((_DIM, _S), jnp.float32),
        ],
        compiler_params=pltpu.CompilerParams(
            dimension_semantics=("parallel", "arbitrary"),
            vmem_limit_bytes=60000 * 1024,
        ),
        name="memory_fused",
        interpret=interpret,
    )(xr, wscore, wqk, wwv, wrvT, wm8)
    return out.reshape(_B, _T, _DIM), memT.transpose(0, 2, 1)


@jax.jit
def kernel(x, memory_slots, W_read_q, W_read_kv, W_write_q, W_write_k,
           W_write_v, write_matter):
    return _run(x, memory_slots, W_read_q, W_read_kv, W_write_q, W_write_k,
                W_write_v, write_matter)
